# Initial kernel scaffold; baseline (speedup 1.0000x reference)
import jax
import jax.numpy as jnp
from jax.experimental import pallas as pl


def kernel(inp, enc_w, enc_b, head_w, head_b):
    raise NotImplementedError("write your pallas kernel here")


# speedup vs baseline: 219.4618x; 219.4618x over previous
import functools
import math

import jax
import jax.numpy as jnp
from jax.experimental import pallas as pl

B, H, W = 2, 224, 224
HW = H * W
GAUSS_DIM = 8
HEAD_IN = 64
R = 4

_INTERPRET = False

_PBLK = 6272
_NPB = HW // _PBLK

_RBLK = 32
_NRB = H // _RBLK
_PH, _PW = 240, 256


def _params_kernel(x_ref, w1_ref, b1_ref, w2_ref, b2_ref, out_ref):
    x = x_ref[0]
    w1 = w1_ref[...]
    w2 = w2_ref[...]
    feat = jax.lax.dot_general(w1, x, (((1,), (0,)), ((), ())),
                               preferred_element_type=jnp.float32)
    feat = jnp.maximum(feat + b1_ref[...], 0.0)
    pred = jax.lax.dot_general(w2, feat, (((1,), (0,)), ((), ())),
                               preferred_element_type=jnp.float32)
    pred = pred + b2_ref[...]

    rgb = pred[0:3]
    sg = 1.0 / (1.0 + jnp.exp(-pred[3:6]))
    theta = sg[0] * (2.0 * math.pi)
    sx = (sg[1] * 0.5 + 1e-6) * (W * 0.5)
    sy = (sg[2] * 0.5 + 1e-6) * (H * 0.5)
    off = jnp.tanh(pred[6:8])
    ct = jnp.cos(theta)
    st = jnp.sin(theta)
    sx2 = sx * sx
    sy2 = sy * sy
    ct2 = ct * ct
    st2 = st * st
    a = ct2 * sx2 + st2 * sy2
    bcov = ct * st * (sx2 - sy2)
    c = st2 * sx2 + ct2 * sy2
    det = a * c - bcov * bcov + 1e-12
    inv = 1.0 / det
    cA = c * inv
    cB = -bcov * inv
    cC = a * inv
    fx0 = 1.0 - off[0]
    fy0 = 1.0 - off[1]
    icx = -jnp.floor(off[0] - 0.5)
    icy = -jnp.floor(off[1] - 0.5)
    out_ref[0] = jnp.concatenate(
        [fx0[None], fy0[None], icx[None], icy[None],
         cA[None], cB[None], cC[None], rgb], axis=0)


def _splat_kernel(pl_ref, out_ref):
    zero = jnp.zeros((_RBLK, W), jnp.float32)
    accr, accg, accb = zero, zero, zero
    for da in range(11):
        da_f = float(da - 4)
        slab = pl_ref[0, 0, :, da:da + _RBLK, :]
        for dbs in range(11):
            db_f = float(dbs - 4)
            s = slab[:, :, dbs:dbs + W]
            fx = s[0] - db_f
            fy = s[1] - da_f
            win = ((jnp.abs(s[2] - db_f) <= 4.0)
                   & (jnp.abs(s[3] - da_f) <= 4.0))
            power = (-0.5 * (s[4] * fx * fx + s[6] * fy * fy)
                     - s[5] * fx * fy)
            alpha = jnp.exp(jnp.minimum(power, 0.0))
            wgt = jnp.where(win, alpha, 0.0)
            accr = accr + wgt * s[7]
            accg = accg + wgt * s[8]
            accb = accb + wgt * s[9]
    img = jnp.stack([accr, accg, accb], axis=0)
    out_ref[0] = jnp.clip(img, 0.0, 1.0)


@jax.jit
def kernel(inp, enc_w, enc_b, head_w, head_b):
    xp = jnp.pad(inp, ((0, 0), (0, 0), (1, 1), (1, 1)))
    slabs = [xp[:, :, dy:dy + H, dx:dx + W]
             for dy in range(3) for dx in range(3)]
    x = jnp.stack(slabs, axis=2).reshape(B, 27, HW)
    x = jnp.pad(x, ((0, 0), (0, 5), (0, 0)))
    w1 = jnp.pad(enc_w.reshape(HEAD_IN, 27), ((0, 0), (0, 5)))
    w2 = head_w.reshape(GAUSS_DIM, HEAD_IN)
    b1 = enc_b.reshape(HEAD_IN, 1)
    b2 = head_b.reshape(GAUSS_DIM, 1)

    planes = pl.pallas_call(
        _params_kernel,
        grid=(B, _NPB),
        in_specs=[
            pl.BlockSpec((1, 32, _PBLK), lambda b, p: (b, 0, p)),
            pl.BlockSpec((HEAD_IN, 32), lambda b, p: (0, 0)),
            pl.BlockSpec((HEAD_IN, 1), lambda b, p: (0, 0)),
            pl.BlockSpec((GAUSS_DIM, HEAD_IN), lambda b, p: (0, 0)),
            pl.BlockSpec((GAUSS_DIM, 1), lambda b, p: (0, 0)),
        ],
        out_specs=pl.BlockSpec((1, 10, _PBLK), lambda b, p: (b, 0, p)),
        out_shape=jax.ShapeDtypeStruct((B, 10, HW), jnp.float32),
        interpret=_INTERPRET,
    )(x, w1, b1, w2, b2)

    planes = planes.reshape(B, 10, H, W)
    pad_rc = ((0, 0), (0, 0), (4, _PH - H - 4), (4, _PW - W - 4))
    pg0 = jnp.pad(planes[:, 0:2], pad_rc)
    pg1 = jnp.pad(planes[:, 2:4], pad_rc, constant_values=99.0)
    pg2 = jnp.pad(planes[:, 4:10], pad_rc)
    padded = jnp.concatenate([pg0, pg1, pg2], axis=1)
    slabs = jnp.stack([padded[:, :, j * _RBLK:j * _RBLK + _RBLK + 16, :]
                       for j in range(_NRB)], axis=1)

    img = pl.pallas_call(
        _splat_kernel,
        grid=(B, _NRB),
        in_specs=[pl.BlockSpec((1, 1, 10, _RBLK + 16, _PW),
                               lambda b, j: (b, j, 0, 0, 0))],
        out_specs=pl.BlockSpec((1, 3, _RBLK, W), lambda b, j: (b, 0, j, 0)),
        out_shape=jax.ShapeDtypeStruct((B, 3, H, W), jnp.float32),
        interpret=_INTERPRET,
    )(slabs)
    return img
